# baseline (device time: 192098 ns/iter reference)
import jax
import jax.numpy as jnp
from jax import lax
from jax.experimental import pallas as pl
from jax.experimental.pallas import tpu as pltpu

N_DEV = 32
T = 512
TH = T // 2
D = 256
H = 512
NE = 128
E_LOC = 4
C_OFF = D
A_OFF = D + NE
BLK = D + NE + H
LAST = N_DEV - 1


def _build_cycle():
    wmap = {(0, 0): 0, (1, 0): 1, (1, 1): 2, (0, 1): 3,
            (0, 2): 4, (1, 2): 5, (1, 3): 6, (0, 3): 7}
    P = [(0, 0), (1, 0), (2, 0), (3, 0), (3, 1), (2, 1), (1, 1), (0, 1),
         (0, 2), (1, 2), (2, 2), (3, 2), (3, 3), (2, 3), (1, 3), (0, 3)]
    cyc = [(0, y, z) for (y, z) in P] + [(1, y, z) for (y, z) in reversed(P)]
    idx = [8 * z + wmap[(x, y)] for (x, y, z) in cyc]
    assert sorted(idx) == list(range(N_DEV))
    nxt = [0] * N_DEV
    prv = [0] * N_DEV
    for i in range(N_DEV):
        a, b = idx[i], idx[(i + 1) % N_DEV]
        nxt[a] = b
        prv[b] = a
    return nxt, prv


_NEXT, _PREV = _build_cycle()


def kernel(x, router_W, route_idx, expert_W):
    H2 = H // 2

    def body(x_ref, rw_ref, idx_ref, ew_ref, nxt_ref, prv_ref, out_ref,
             comm_a, comm_b, w2a_ref, w2b_ref,
             xc_send_a, xc_recv_a, ac1_send_a, ac1_recv_a,
             ac2_send_a, ac2_recv_a,
             xc_send_b, xc_recv_b, ac1_send_b, ac1_recv_b,
             ac2_send_b, ac2_recv_b,
             xc_cred_a, ac_cred_a, xc_cred_b, ac_cred_b):
        my = lax.axis_index("i")
        nxt = nxt_ref[my]
        prv = prv_ref[my]

        barrier_sem = pltpu.get_barrier_semaphore()
        for nbr in (prv, nxt):
            pl.semaphore_signal(
                barrier_sem, inc=1,
                device_id=(nbr,), device_id_type=pl.DeviceIdType.MESH,
            )
        pl.semaphore_wait(barrier_sem, 2)

        scores = jnp.dot(x_ref[...], rw_ref[...],
                         preferred_element_type=jnp.float32)
        lane = lax.broadcasted_iota(jnp.int32, (T, NE), 1)
        e0 = idx_ref[:, 0:1]
        e1 = idx_ref[:, 1:2]
        oh0 = (lane == e0).astype(jnp.float32)
        oh1 = (lane == e1).astype(jnp.float32)
        s0 = jnp.sum(scores * oh0, axis=1, keepdims=True)
        s1 = jnp.sum(scores * oh1, axis=1, keepdims=True)
        w0 = jax.nn.sigmoid(s0 - s1)
        c = w0 * oh0 + (1.0 - w0) * oh1

        for comm, lo in ((comm_a, 0), (comm_b, TH)):
            comm[0, :, 0:C_OFF] = x_ref[lo:lo + TH, :].astype(jnp.bfloat16)
            comm[0, :, C_OFF:A_OFF] = c[lo:lo + TH, :].astype(jnp.bfloat16)
            comm[0, :, A_OFF:BLK] = jnp.zeros((TH, H), jnp.bfloat16)

        for k in range(E_LOC):
            w2a_ref[:, k * H2:(k + 1) * H2] = ew_ref[k][:, 0:H2].astype(
                jnp.bfloat16)
            w2b_ref[:, k * H2:(k + 1) * H2] = ew_ref[k][:, H2:H].astype(
                jnp.bfloat16)

        row = lax.broadcasted_iota(jnp.int32, (NE, E_LOC), 0)
        col = lax.broadcasted_iota(jnp.int32, (NE, E_LOC), 1)
        sel = (row == E_LOC * my + col).astype(jnp.float32)

        rings = {
            "a": dict(comm=comm_a, xc_send=xc_send_a, xc_recv=xc_recv_a,
                      ac1_send=ac1_send_a, ac1_recv=ac1_recv_a,
                      ac2_send=ac2_send_a, ac2_recv=ac2_recv_a,
                      xc_cred=xc_cred_a, ac_cred=ac_cred_a,
                      down=nxt, up=prv, prev=None),
            "b": dict(comm=comm_b, xc_send=xc_send_b, xc_recv=xc_recv_b,
                      ac1_send=ac1_send_b, ac1_recv=ac1_recv_b,
                      ac2_send=ac2_send_b, ac2_recv=ac2_recv_b,
                      xc_cred=xc_cred_b, ac_cred=ac_cred_b,
                      down=prv, up=nxt, prev=None),
        }

        def xc_ref(comm, slot):
            return comm.at[slot, :, pl.ds(0, A_OFF)]

        def ac1_ref(comm, slot):
            return comm.at[slot, :, pl.ds(A_OFF, H2)]

        def ac2_ref(comm, slot):
            return comm.at[slot, :, pl.ds(A_OFF + H2, H2)]

        def cleanup(R, h):
            prev_xc, prev_ac1, prev_ac2 = R["prev"]
            if prev_xc is not None:
                prev_xc.wait_send()
                if h - 1 <= LAST - 2:
                    pl.semaphore_signal(
                        R["xc_cred"], inc=1,
                        device_id=(R["up"],), device_id_type=pl.DeviceIdType.MESH,
                    )
            prev_ac1.wait_send()
            prev_ac2.wait_send()
            if h - 1 <= N_DEV - 2:
                pl.semaphore_signal(
                    R["ac_cred"], inc=1,
                    device_id=(R["up"],), device_id_type=pl.DeviceIdType.MESH,
                )

        def xc_phase(R, h, s, r):
            comm = R["comm"]
            if h >= 1:
                rcv = pltpu.make_async_remote_copy(
                    src_ref=xc_ref(comm, s), dst_ref=xc_ref(comm, s),
                    send_sem=R["xc_send"].at[h - 1],
                    recv_sem=R["xc_recv"].at[h - 1],
                    device_id=(R["up"],), device_id_type=pl.DeviceIdType.MESH,
                )
                rcv.wait_recv()
            if h <= LAST - 1:
                if h >= 1:
                    pl.semaphore_wait(R["xc_cred"], 1)
                fwd = pltpu.make_async_remote_copy(
                    src_ref=xc_ref(comm, s), dst_ref=xc_ref(comm, r),
                    send_sem=R["xc_send"].at[h],
                    recv_sem=R["xc_recv"].at[h],
                    device_id=(R["down"],), device_id_type=pl.DeviceIdType.MESH,
                )
                fwd.start()
            else:
                fwd = None
            return fwd

        def ac_chunk(R, h, s, r, which, contrib, lo):
            comm = R["comm"]
            cref = ac1_ref if which == 1 else ac2_ref
            send = R[f"ac{which}_send"]
            recv = R[f"ac{which}_recv"]
            if h >= 1:
                rcv = pltpu.make_async_remote_copy(
                    src_ref=cref(comm, s), dst_ref=cref(comm, s),
                    send_sem=send.at[h - 1], recv_sem=recv.at[h - 1],
                    device_id=(R["up"],), device_id_type=pl.DeviceIdType.MESH,
                )
                rcv.wait_recv()
            acc = comm[s, :, lo:lo + H2].astype(jnp.float32) + contrib
            comm[s, :, lo:lo + H2] = acc.astype(jnp.bfloat16)
            if which == 1 and h >= 1:
                pl.semaphore_wait(R["ac_cred"], 1)
            snd = pltpu.make_async_remote_copy(
                src_ref=cref(comm, s), dst_ref=cref(comm, r),
                send_sem=send.at[h], recv_sem=recv.at[h],
                device_id=(R["down"],), device_id_type=pl.DeviceIdType.MESH,
            )
            snd.start()
            return snd

        def ac_phase(R, h, s, r):
            comm = R["comm"]
            x_b = comm[s, :, 0:C_OFF]
            c_b = comm[s, :, C_OFF:A_OFF]
            coef = jnp.dot(c_b, sel, preferred_element_type=jnp.float32)

            def contrib_half(w2):
                y = jnp.dot(x_b, w2[...], preferred_element_type=jnp.float32)
                out = coef[:, 0:1] * y[:, 0:H2]
                for k in range(1, E_LOC):
                    out = out + coef[:, k:k + 1] * y[:, k * H2:(k + 1) * H2]
                return out

            snd1 = ac_chunk(R, h, s, r, 1, contrib_half(w2a_ref), A_OFF)
            snd2 = ac_chunk(R, h, s, r, 2, contrib_half(w2b_ref), A_OFF + H2)
            return snd1, snd2

        for h in range(N_DEV):
            s = h % 2
            r = (h + 1) % 2
            for R in (rings["a"], rings["b"]):
                if h >= 1:
                    cleanup(R, h)
                fwd = xc_phase(R, h, s, r)
                snd1, snd2 = ac_phase(R, h, s, r)
                R["prev"] = (fwd, snd1, snd2)

        for R in (rings["a"], rings["b"]):
            comm = R["comm"]
            R["prev"][1].wait_send()
            R["prev"][2].wait_send()
            for which, cref in ((1, ac1_ref), (2, ac2_ref)):
                fin = pltpu.make_async_remote_copy(
                    src_ref=cref(comm, 0), dst_ref=cref(comm, 0),
                    send_sem=R[f"ac{which}_send"].at[LAST],
                    recv_sem=R[f"ac{which}_recv"].at[LAST],
                    device_id=(R["up"],), device_id_type=pl.DeviceIdType.MESH,
                )
                fin.wait_recv()

        out_ref[0:TH, :] = comm_a[0, :, A_OFF:BLK].astype(jnp.float32)
        out_ref[TH:T, :] = comm_b[0, :, A_OFF:BLK].astype(jnp.float32)

    dma = pltpu.SemaphoreType.DMA((N_DEV,))
    return pl.pallas_call(
        body,
        out_shape=jax.ShapeDtypeStruct((T, H), jnp.float32),
        in_specs=[pl.BlockSpec(memory_space=pltpu.VMEM)] * 4
        + [pl.BlockSpec(memory_space=pltpu.SMEM)] * 2,
        out_specs=pl.BlockSpec(memory_space=pltpu.VMEM),
        scratch_shapes=[
            pltpu.VMEM((2, TH, BLK), jnp.bfloat16),
            pltpu.VMEM((2, TH, BLK), jnp.bfloat16),
            pltpu.VMEM((D, E_LOC * H // 2), jnp.bfloat16),
            pltpu.VMEM((D, E_LOC * H // 2), jnp.bfloat16),
            dma, dma, dma, dma, dma, dma,
            dma, dma, dma, dma, dma, dma,
            pltpu.SemaphoreType.REGULAR,
            pltpu.SemaphoreType.REGULAR,
            pltpu.SemaphoreType.REGULAR,
            pltpu.SemaphoreType.REGULAR,
        ],
        compiler_params=pltpu.CompilerParams(collective_id=0),
    )(x, router_W, route_idx, expert_W,
      jnp.asarray(_NEXT, jnp.int32), jnp.asarray(_PREV, jnp.int32))
